# Initial kernel scaffold; baseline (speedup 1.0000x reference)
import jax
import jax.numpy as jnp
from jax.experimental import pallas as pl


def kernel(table, idx, targets, actions):
    raise NotImplementedError("write your pallas kernel here")


# speedup vs baseline: 3.2054x; 3.2054x over previous
import jax
import jax.numpy as jnp
from jax import lax
from jax.experimental import pallas as pl
from jax.experimental.pallas import tpu as pltpu
from jax.experimental.pallas import tpu_sc as plsc

LEARN_RATE = 0.2
_NC, _NS, _L = 2, 16, 16
_NW = _NC * _NS


def _sc_body(newtab_ref, table_ref, idx_ref, tgt_ref, act_ref, loss_ref,
             idxv, actv, tgtv, fidxv, qv, nqv, lossv, sem):
    nch = idxv.shape[0]
    wid = lax.axis_index("s") * _NC + lax.axis_index("c")
    base = wid * nch
    pltpu.sync_copy(idx_ref.at[pl.ds(base, nch)], idxv)
    pltpu.sync_copy(act_ref.at[pl.ds(base, nch)], actv)
    pltpu.sync_copy(tgt_ref.at[pl.ds(base, nch)], tgtv)
    nacts = 16
    for j in range(nch):
        for k in range(128 // _L):
            s = pl.ds(k * _L, _L)
            fidxv[j, s] = idxv[j, s] * nacts + actv[j, s]
    gathers = [
        pltpu.async_copy(table_ref.at[fidxv.at[j]], qv.at[j], sem)
        for j in range(nch)
    ]
    for g in gathers:
        g.wait()
    acc = jnp.zeros((_L,), jnp.float32)
    for j in range(nch):
        for k in range(128 // _L):
            s = pl.ds(k * _L, _L)
            q = qv[j, s]
            d = tgtv[j, s] - q
            nqv[j, s] = q + LEARN_RATE * d
            acc = acc + d * d
    lossv[...] = acc
    scatters = [
        pltpu.async_copy(nqv.at[j], newtab_ref.at[fidxv.at[j]], sem)
        for j in range(nch)
    ]
    for c in scatters:
        c.wait()
    pltpu.sync_copy(lossv, loss_ref.at[wid])


def kernel(table, idx, targets, actions):
    batch = idx.shape[0]
    tflat = table.reshape(-1)
    rows = batch // 128
    nch = rows // _NW
    idx2 = idx.reshape(rows, 128)
    act2 = actions.reshape(rows, 128)
    tgt2 = targets.reshape(rows, 128)

    mesh = plsc.VectorSubcoreMesh(
        core_axis_name="c", subcore_axis_name="s",
        num_cores=_NC, num_subcores=_NS)
    sck = pl.kernel(
        _sc_body,
        out_type=jax.ShapeDtypeStruct((_NW, _L), jnp.float32),
        mesh=mesh,
        scratch_types=[
            pltpu.VMEM((nch, 128), jnp.int32),
            pltpu.VMEM((nch, 128), jnp.int32),
            pltpu.VMEM((nch, 128), jnp.float32),
            pltpu.VMEM((nch, 128), jnp.int32),
            pltpu.VMEM((nch, 128), jnp.float32),
            pltpu.VMEM((nch, 128), jnp.float32),
            pltpu.VMEM((_L,), jnp.float32),
            pltpu.SemaphoreType.DMA,
        ],
    )
    newtab = jax.new_ref(tflat)
    loss_part = sck(newtab, tflat, idx2, tgt2, act2)
    new_table = newtab[...].reshape(table.shape)
    loss = jnp.sum(loss_part)
    return new_table, loss
